# baseline (speedup 1.0000x reference)
import functools

import jax
import jax.numpy as jnp
from jax import lax
from jax.experimental import pallas as pl
from jax.experimental.pallas import tpu as pltpu
from jax.experimental.pallas import tpu_sc as plsc

_N = 10000
_NNZ = 320000
_FIN = 128
_FOUT = 128

_NC = 2
_NS = 16
_NW = _NC * _NS
_L = 16
_EPW = _NNZ // _NW
_NV = _EPW // _L
_NB = _N // _L
_NP = 10240
_OUT = _NW * 3 * _NP
_GIW = 10112
_RB = _NP // 128


def _rowsum_body(x_ref, s_ref):
    s_ref[...] = jnp.sum(x_ref[...], axis=1)


def _sc_edges_body(s_hbm, gi_hbm, val_hbm, out_hbm,
                   s_v, rc_v, val_v, acc_v, sem):
    wid = lax.axis_index("s") * _NC + lax.axis_index("c")
    base = wid * _EPW
    off = lax.rem(base, 128)
    start = pl.multiple_of(base - off, 128)

    c1 = pltpu.async_copy(s_hbm, s_v, sem)
    c2 = pltpu.async_copy(gi_hbm.at[:, pl.ds(start, _GIW)], rc_v, sem)
    c3 = pltpu.async_copy(val_hbm.at[pl.ds(base, _EPW)], val_v, sem)

    zeros = jnp.zeros((_L,), jnp.float32)
    nsplat = jnp.full((_L,), _N, jnp.int32)
    nsplat2 = jnp.full((_L,), 2 * _N, jnp.int32)

    def _zero(i, _):
        acc_v[pl.ds(i * _L, _L)] = zeros
        return 0

    lax.fori_loop(0, 3 * _NB, _zero, 0, unroll=4)
    c1.wait()
    c2.wait()
    c3.wait()

    def _pass1(i, _):
        sl = pl.ds(off + i * _L, _L)
        r = rc_v[0, sl]
        c = rc_v[1, sl]
        v = val_v[pl.ds(i * _L, _L)]
        g = plsc.load_gather(s_v, [c])
        pv = v * g
        v2 = v * v
        p2 = v2 * g
        p6 = v2 * v2 * p2
        plsc.addupdate_scatter(acc_v, [r], pv)
        plsc.addupdate_scatter(acc_v, [r + nsplat], p2)
        plsc.addupdate_scatter(acc_v, [r + nsplat2], p6)
        return 0

    obase = wid * (3 * _NP)
    lax.fori_loop(0, _NV, _pass1, 0, unroll=2)
    pltpu.sync_copy(acc_v.at[pl.ds(0, _N)], out_hbm.at[pl.ds(obase, _N)])
    pltpu.sync_copy(acc_v.at[pl.ds(_N, _N)],
                    out_hbm.at[pl.ds(obase + _NP, _N)])
    pltpu.sync_copy(acc_v.at[pl.ds(2 * _N, _N)],
                    out_hbm.at[pl.ds(obase + 2 * _NP, _N)])


def _final_body(p_ref, fc_ref, y_ref):
    t = jnp.sum(p_ref[...], axis=0)
    t24 = t.reshape(24, 128)
    tt = jnp.transpose(t24)
    fc = fc_ref[...]
    fct = jnp.transpose(fc)
    w = jnp.concatenate(
        [fct[0:1] + fct[1:2], fct[2:3], fct[3:4]], axis=0)
    for rr in range(8):
        z = jnp.concatenate(
            [tt[:, rr:rr + 1],
             tt[:, 8 + rr:8 + rr + 1],
             tt[:, 16 + rr:16 + rr + 1]], axis=1)
        blk = jax.lax.dot_general(
            z, w, (((1,), (0,)), ((), ())),
            preferred_element_type=jnp.float32,
            precision=jax.lax.Precision.HIGHEST)
        y_ref[pl.ds(rr * 128, 128), :] = blk


@jax.jit
def kernel(x, gso_indices, gso_values, filterCoeff):
    s = pl.pallas_call(
        _rowsum_body,
        grid=(5,),
        in_specs=[pl.BlockSpec((2048, _FIN), lambda i: (i, 0))],
        out_specs=pl.BlockSpec((2048,), lambda i: (i,)),
        out_shape=jax.ShapeDtypeStruct((_NP,), jnp.float32),
    )(x)

    sc_edges = functools.partial(
        pl.kernel,
        mesh=plsc.VectorSubcoreMesh(core_axis_name="c", subcore_axis_name="s"),
        out_type=jax.ShapeDtypeStruct((_OUT,), jnp.float32),
        compiler_params=pltpu.CompilerParams(needs_layout_passes=False),
        scratch_types=[
            pltpu.VMEM((_NP,), jnp.float32),
            pltpu.VMEM((2, _GIW), jnp.int32),
            pltpu.VMEM((_EPW,), jnp.float32),
            pltpu.VMEM((3 * _N,), jnp.float32),
            pltpu.SemaphoreType.DMA,
        ],
    )(_sc_edges_body)
    partials = sc_edges(s, gso_indices, gso_values)
    p4 = partials.reshape(_NW, 3, _RB, 128)

    y = pl.pallas_call(
        _final_body,
        grid=(10,),
        in_specs=[pl.BlockSpec((_NW, 3, 8, 128), lambda i: (0, 0, i, 0)),
                  pl.BlockSpec((_FOUT, 4), lambda i: (0, 0))],
        out_specs=pl.BlockSpec((1024, _FOUT), lambda i: (i, 0)),
        out_shape=jax.ShapeDtypeStruct((_N, _FOUT), jnp.float32),
    )(p4, filterCoeff)
    return y


# speedup vs baseline: 1.0607x; 1.0607x over previous
import functools

import jax
import jax.numpy as jnp
from jax import lax
from jax.experimental import pallas as pl
from jax.experimental.pallas import tpu as pltpu
from jax.experimental.pallas import tpu_sc as plsc

_N = 10000
_NNZ = 320000
_FIN = 128
_FOUT = 128

_NC = 2
_NS = 16
_NW = _NC * _NS
_L = 16
_EPW = _NNZ // _NW
_NV = _EPW // _L
_NB = _N // _L
_NP = 10240
_OUT = _NW * 3 * _NP
_GIW = 10112
_RB = _NP // 128


def _rowsum_body(x_ref, s_ref):
    s_ref[...] = jnp.sum(x_ref[...], axis=1)


def _sc_edges_body(s_hbm, gi_hbm, val_hbm, out_hbm,
                   s_v, rc_v, val_v, acc_v, sem):
    wid = lax.axis_index("s") * _NC + lax.axis_index("c")
    base = wid * _EPW
    off = lax.rem(base, 128)
    start = pl.multiple_of(base - off, 128)

    c1 = pltpu.async_copy(s_hbm, s_v, sem)
    c2 = pltpu.async_copy(gi_hbm.at[:, pl.ds(start, _GIW)], rc_v, sem)
    c3 = pltpu.async_copy(val_hbm.at[pl.ds(base, _EPW)], val_v, sem)

    zeros = jnp.zeros((_L,), jnp.float32)
    nsplat = jnp.full((_L,), _N, jnp.int32)
    nsplat2 = jnp.full((_L,), 2 * _N, jnp.int32)

    def _zero(i, _):
        acc_v[pl.ds(i * _L, _L)] = zeros
        return 0

    lax.fori_loop(0, 3 * _NB, _zero, 0, unroll=4)
    c1.wait()
    c2.wait()
    c3.wait()

    def _pass1(i, _):
        sl = pl.ds(off + i * _L, _L)
        r = rc_v[0, sl]
        c = rc_v[1, sl]
        v = val_v[pl.ds(i * _L, _L)]
        g = plsc.load_gather(s_v, [c])
        pv = v * g
        v2 = v * v
        p2 = v2 * g
        p6 = v2 * v2 * p2
        plsc.addupdate_scatter(acc_v, [r], pv)
        plsc.addupdate_scatter(acc_v, [r + nsplat], p2)
        plsc.addupdate_scatter(acc_v, [r + nsplat2], p6)
        return 0

    obase = wid * (3 * _NP)
    lax.fori_loop(0, _NV, _pass1, 0, unroll=2)
    pltpu.sync_copy(acc_v.at[pl.ds(0, _N)], out_hbm.at[pl.ds(obase, _N)])
    pltpu.sync_copy(acc_v.at[pl.ds(_N, _N)],
                    out_hbm.at[pl.ds(obase + _NP, _N)])
    pltpu.sync_copy(acc_v.at[pl.ds(2 * _N, _N)],
                    out_hbm.at[pl.ds(obase + 2 * _NP, _N)])


def _final_body(p_ref, fc_ref, y_ref):
    t = jnp.sum(p_ref[...], axis=0)
    t24 = t.reshape(48, 128)
    tt = jnp.transpose(t24)
    fc = fc_ref[...]
    fct = jnp.transpose(fc)
    w = jnp.concatenate(
        [fct[0:1] + fct[1:2], fct[2:3], fct[3:4]], axis=0)
    for rr in range(16):
        z = jnp.concatenate(
            [tt[:, rr:rr + 1],
             tt[:, 16 + rr:16 + rr + 1],
             tt[:, 32 + rr:32 + rr + 1]], axis=1)
        blk = jax.lax.dot_general(
            z, w, (((1,), (0,)), ((), ())),
            preferred_element_type=jnp.float32,
            precision=jax.lax.Precision.HIGHEST)
        y_ref[pl.ds(rr * 128, 128), :] = blk


@jax.jit
def kernel(x, gso_indices, gso_values, filterCoeff):
    s = pl.pallas_call(
        _rowsum_body,
        grid=(5,),
        in_specs=[pl.BlockSpec((2048, _FIN), lambda i: (i, 0))],
        out_specs=pl.BlockSpec((2048,), lambda i: (i,)),
        out_shape=jax.ShapeDtypeStruct((_NP,), jnp.float32),
    )(x)

    sc_edges = functools.partial(
        pl.kernel,
        mesh=plsc.VectorSubcoreMesh(core_axis_name="c", subcore_axis_name="s"),
        out_type=jax.ShapeDtypeStruct((_OUT,), jnp.float32),
        compiler_params=pltpu.CompilerParams(needs_layout_passes=False),
        scratch_types=[
            pltpu.VMEM((_NP,), jnp.float32),
            pltpu.VMEM((2, _GIW), jnp.int32),
            pltpu.VMEM((_EPW,), jnp.float32),
            pltpu.VMEM((3 * _N,), jnp.float32),
            pltpu.SemaphoreType.DMA,
        ],
    )(_sc_edges_body)
    partials = sc_edges(s, gso_indices, gso_values)
    p4 = partials.reshape(_NW, 3, _RB, 128)

    y = pl.pallas_call(
        _final_body,
        grid=(5,),
        in_specs=[pl.BlockSpec((_NW, 3, 16, 128), lambda i: (0, 0, i, 0)),
                  pl.BlockSpec((_FOUT, 4), lambda i: (0, 0))],
        out_specs=pl.BlockSpec((2048, _FOUT), lambda i: (i, 0)),
        out_shape=jax.ShapeDtypeStruct((_N, _FOUT), jnp.float32),
    )(p4, filterCoeff)
    return y


# speedup vs baseline: 1.0719x; 1.0105x over previous
import functools

import jax
import jax.numpy as jnp
from jax import lax
from jax.experimental import pallas as pl
from jax.experimental.pallas import tpu as pltpu
from jax.experimental.pallas import tpu_sc as plsc

_N = 10000
_NNZ = 320000
_FIN = 128
_FOUT = 128

_NC = 2
_NS = 16
_NW = _NC * _NS
_L = 16
_EPW = _NNZ // _NW
_NV = _EPW // _L
_NB = _N // _L
_NP = 10240
_OUT = _NW * 3 * _NP
_GIW = 10112
_RB = _NP // 128


def _rowsum_body(x_ref, s_ref):
    s_ref[...] = jnp.sum(x_ref[...], axis=1)


def _sc_edges_body(s_hbm, gi_hbm, val_hbm, out_hbm,
                   s_v, rc_v, val_v, acc_v, sem):
    wid = lax.axis_index("s") * _NC + lax.axis_index("c")
    base = wid * _EPW
    off = lax.rem(base, 128)
    start = pl.multiple_of(base - off, 128)

    c1 = pltpu.async_copy(s_hbm, s_v, sem)
    c2 = pltpu.async_copy(gi_hbm.at[:, pl.ds(start, _GIW)], rc_v, sem)
    c3 = pltpu.async_copy(val_hbm.at[pl.ds(base, _EPW)], val_v, sem)

    zeros = jnp.zeros((_L,), jnp.float32)
    nsplat = jnp.full((_L,), _N, jnp.int32)
    nsplat2 = jnp.full((_L,), 2 * _N, jnp.int32)

    def _zero(i, _):
        acc_v[pl.ds(i * _L, _L)] = zeros
        return 0

    lax.fori_loop(0, 3 * _NB, _zero, 0, unroll=4)
    c1.wait()
    c2.wait()
    c3.wait()

    def _pass1(i, _):
        sl = pl.ds(off + i * _L, _L)
        r = rc_v[0, sl]
        c = rc_v[1, sl]
        v = val_v[pl.ds(i * _L, _L)]
        g = plsc.load_gather(s_v, [c])
        pv = v * g
        v2 = v * v
        p2 = v2 * g
        p6 = v2 * v2 * p2
        plsc.addupdate_scatter(acc_v, [r], pv)
        plsc.addupdate_scatter(acc_v, [r + nsplat], p2)
        plsc.addupdate_scatter(acc_v, [r + nsplat2], p6)
        return 0

    obase = wid * (3 * _NP)
    lax.fori_loop(0, _NV, _pass1, 0, unroll=2)
    pltpu.sync_copy(acc_v.at[pl.ds(0, _N)], out_hbm.at[pl.ds(obase, _N)])
    pltpu.sync_copy(acc_v.at[pl.ds(_N, _N)],
                    out_hbm.at[pl.ds(obase + _NP, _N)])
    pltpu.sync_copy(acc_v.at[pl.ds(2 * _N, _N)],
                    out_hbm.at[pl.ds(obase + 2 * _NP, _N)])


def _final_body(p_ref, fc_ref, y_ref):
    t = jnp.sum(p_ref[...], axis=0)
    t24 = t.reshape(48, 128)
    tt = jnp.transpose(t24)
    fc = fc_ref[...]
    fct = jnp.transpose(fc)
    w = jnp.concatenate(
        [fct[0:1] + fct[1:2], fct[2:3], fct[3:4]], axis=0)
    for rr in range(16):
        z = jnp.concatenate(
            [tt[:, rr:rr + 1],
             tt[:, 16 + rr:16 + rr + 1],
             tt[:, 32 + rr:32 + rr + 1]], axis=1)
        blk = jax.lax.dot_general(
            z, w, (((1,), (0,)), ((), ())),
            preferred_element_type=jnp.float32,
            precision=jax.lax.Precision.HIGHEST)
        y_ref[pl.ds(rr * 128, 128), :] = blk


@jax.jit
def kernel(x, gso_indices, gso_values, filterCoeff):
    s = pl.pallas_call(
        _rowsum_body,
        grid=(2,),
        in_specs=[pl.BlockSpec((5120, _FIN), lambda i: (i, 0))],
        out_specs=pl.BlockSpec((5120,), lambda i: (i,)),
        out_shape=jax.ShapeDtypeStruct((_NP,), jnp.float32),
    )(x)

    sc_edges = functools.partial(
        pl.kernel,
        mesh=plsc.VectorSubcoreMesh(core_axis_name="c", subcore_axis_name="s"),
        out_type=jax.ShapeDtypeStruct((_OUT,), jnp.float32),
        compiler_params=pltpu.CompilerParams(needs_layout_passes=False),
        scratch_types=[
            pltpu.VMEM((_NP,), jnp.float32),
            pltpu.VMEM((2, _GIW), jnp.int32),
            pltpu.VMEM((_EPW,), jnp.float32),
            pltpu.VMEM((3 * _N,), jnp.float32),
            pltpu.SemaphoreType.DMA,
        ],
    )(_sc_edges_body)
    partials = sc_edges(s, gso_indices, gso_values)
    p4 = partials.reshape(_NW, 3, _RB, 128)

    y = pl.pallas_call(
        _final_body,
        grid=(5,),
        in_specs=[pl.BlockSpec((_NW, 3, 16, 128), lambda i: (0, 0, i, 0)),
                  pl.BlockSpec((_FOUT, 4), lambda i: (0, 0))],
        out_specs=pl.BlockSpec((2048, _FOUT), lambda i: (i, 0)),
        out_shape=jax.ShapeDtypeStruct((_N, _FOUT), jnp.float32),
    )(p4, filterCoeff)
    return y
